# Initial kernel scaffold; baseline (speedup 1.0000x reference)
import functools

import jax
import jax.numpy as jnp
from jax import lax
from jax.experimental import pallas as pl
from jax.experimental.pallas import tpu as pltpu
from jax.experimental.pallas import tpu_sc as plsc

N = 10000
E = 320000
D = 128
NC, NS, L = 2, 16, 16
NW = NC * NS
EPT = E // NW
ROWS_PT = N // NS
K = 400
NCH = EPT // K

_MESH = plsc.VectorSubcoreMesh(core_axis_name="c", subcore_axis_name="s")
_PREC = lax.Precision.HIGHEST


def _worker(base_len):
    cid = lax.axis_index("c")
    sid = lax.axis_index("s")
    wid = sid * NC + cid
    return cid, sid, wid * base_len


def _zero_shared_rows(z_v, acc_sh, sid):
    nz = z_v.shape[0]
    flat = 1
    for s in z_v.shape:
        flat *= s
    zv2 = z_v.reshape((flat,)) if len(z_v.shape) > 1 else z_v

    @pl.loop(0, flat, step=L)
    def _(i):
        zv2[pl.ds(i, L)] = jnp.zeros((L,), jnp.float32)

    start = sid * ROWS_PT
    off = 0
    while off < ROWS_PT:
        step = min(nz, ROWS_PT - off)
        pltpu.sync_copy(z_v.at[pl.ds(0, step)], acc_sh.at[pl.ds(start + off, step)])
        off += step


@functools.partial(
    pl.kernel,
    out_type=jax.ShapeDtypeStruct((NC, N), jnp.float32),
    mesh=_MESH,
    scratch_types=[
        pltpu.VMEM_SHARED((N,), jnp.float32),
        pltpu.VMEM((EPT,), jnp.float32),
        pltpu.VMEM((EPT,), jnp.int32),
        pltpu.VMEM((ROWS_PT,), jnp.float32),
    ],
)
def _sc_deg(c_hbm, w_hbm, out_hbm, acc_sh, w_v, c_v, z_v):
    cid, sid, base = _worker(EPT)
    _zero_shared_rows(z_v, acc_sh, sid)
    plsc.subcore_barrier()
    pltpu.sync_copy(w_hbm.at[pl.ds(base, EPT)], w_v)
    pltpu.sync_copy(c_hbm.at[pl.ds(base, EPT)], c_v)
    pltpu.sync_copy(w_v, acc_sh.at[c_v], add=True)
    plsc.subcore_barrier()
    s = sid * ROWS_PT
    pltpu.sync_copy(acc_sh.at[pl.ds(s, ROWS_PT)], out_hbm.at[cid, pl.ds(s, ROWS_PT)])


@functools.partial(
    pl.kernel,
    out_type=jax.ShapeDtypeStruct((E,), jnp.float32),
    mesh=_MESH,
    scratch_types=[
        pltpu.VMEM((EPT,), jnp.int32),
        pltpu.VMEM((EPT,), jnp.float32),
        pltpu.VMEM((EPT,), jnp.float32),
        pltpu.SemaphoreType.DMA,
    ],
)
def _sc_wre(r_hbm, ew_hbm, dis_hbm, out_hbm, r_v, d_v, w_v, sem):
    _, _, base = _worker(EPT)
    pltpu.sync_copy(r_hbm.at[pl.ds(base, EPT)], r_v)
    pltpu.sync_copy(ew_hbm.at[pl.ds(base, EPT)], w_v)
    pltpu.async_copy(dis_hbm.at[r_v], d_v, sem).wait()

    @pl.loop(0, EPT, step=L)
    def _(i):
        w_v[pl.ds(i, L)] = w_v[pl.ds(i, L)] * d_v[pl.ds(i, L)]

    pltpu.sync_copy(w_v, out_hbm.at[pl.ds(base, EPT)])


@functools.partial(
    pl.kernel,
    out_type=jax.ShapeDtypeStruct((NC, N, D), jnp.float32),
    mesh=_MESH,
    scratch_types=[
        pltpu.VMEM_SHARED((N, D), jnp.float32),
        pltpu.VMEM((K, D), jnp.float32),
        pltpu.VMEM((K, D), jnp.float32),
        pltpu.VMEM((K,), jnp.int32),
        pltpu.VMEM((K,), jnp.int32),
        pltpu.VMEM((K,), jnp.int32),
        pltpu.VMEM((K,), jnp.int32),
        pltpu.VMEM((K,), jnp.float32),
        pltpu.VMEM((K,), jnp.float32),
        pltpu.SemaphoreType.DMA,
        pltpu.SemaphoreType.DMA,
    ],
)
def _sc_agg(h_hbm, r_hbm, c_hbm, w_hbm, out_hbm, acc_sh,
            rows0, rows1, r0, r1, c0, c1, w0, w1, sem0, sem1):
    cid, sid, base = _worker(EPT)
    _zero_shared_rows(rows0, acc_sh, sid)
    plsc.subcore_barrier()

    bufs = ((rows0, r0, c0, w0, sem0), (rows1, r1, c1, w1, sem1))

    def start(k, buf):
        rows, r_v, c_v, w_v, sem = buf
        off = base + k * K
        pltpu.sync_copy(r_hbm.at[pl.ds(off, K)], r_v)
        pltpu.sync_copy(c_hbm.at[pl.ds(off, K)], c_v)
        pltpu.sync_copy(w_hbm.at[pl.ds(off, K)], w_v)
        return pltpu.async_copy(h_hbm.at[r_v], rows, sem)

    descs = [None, None]
    descs[0] = start(0, bufs[0])
    for k in range(NCH):
        rows, r_v, c_v, w_v, sem = bufs[k % 2]
        descs[k % 2].wait()
        if k + 1 < NCH:
            descs[(k + 1) % 2] = start(k + 1, bufs[(k + 1) % 2])

        @pl.loop(0, K)
        def _(j):
            s = w_v[j]
            for q in range(D // L):
                rows[j, pl.ds(q * L, L)] = rows[j, pl.ds(q * L, L)] * s

        pltpu.sync_copy(rows, acc_sh.at[c_v], add=True)

    plsc.subcore_barrier()
    s0 = sid * ROWS_PT
    pltpu.sync_copy(acc_sh.at[pl.ds(s0, ROWS_PT)],
                    out_hbm.at[cid, pl.ds(s0, ROWS_PT)])


@functools.partial(
    pl.kernel,
    out_type=jax.ShapeDtypeStruct((NC, N), jnp.float32),
    mesh=_MESH,
    scratch_types=[
        pltpu.VMEM_SHARED((N,), jnp.float32),
        pltpu.VMEM((EPT,), jnp.int32),
        pltpu.VMEM((EPT,), jnp.int32),
        pltpu.VMEM((EPT,), jnp.float32),
        pltpu.VMEM((EPT,), jnp.float32),
        pltpu.VMEM((ROWS_PT,), jnp.float32),
        pltpu.SemaphoreType.DMA,
    ],
)
def _sc_agg1(g_hbm, r_hbm, c_hbm, w_hbm, out_hbm, acc_sh,
             r_v, c_v, w_v, g_v, z_v, sem):
    cid, sid, base = _worker(EPT)
    _zero_shared_rows(z_v, acc_sh, sid)
    plsc.subcore_barrier()
    pltpu.sync_copy(r_hbm.at[pl.ds(base, EPT)], r_v)
    pltpu.sync_copy(c_hbm.at[pl.ds(base, EPT)], c_v)
    pltpu.sync_copy(w_hbm.at[pl.ds(base, EPT)], w_v)
    pltpu.async_copy(g_hbm.at[r_v], g_v, sem).wait()

    @pl.loop(0, EPT, step=L)
    def _(i):
        g_v[pl.ds(i, L)] = g_v[pl.ds(i, L)] * w_v[pl.ds(i, L)]

    pltpu.sync_copy(g_v, acc_sh.at[c_v], add=True)
    plsc.subcore_barrier()
    s = sid * ROWS_PT
    pltpu.sync_copy(acc_sh.at[pl.ds(s, ROWS_PT)], out_hbm.at[cid, pl.ds(s, ROWS_PT)])


_BLK = 1000
_GRID = N // _BLK


def _dis_body(deg_ref, dis_ref):
    d = deg_ref[0, :] + deg_ref[1, :] + 1.0
    dis_ref[:, 0] = lax.rsqrt(d)


def _tc_dis(deg_parts):
    return pl.pallas_call(
        _dis_body,
        out_shape=jax.ShapeDtypeStruct((N, 1), jnp.float32),
    )(deg_parts)


def _mm2_body(x_ref, wa_ref, wb_ref, ha_ref, hb_ref):
    x = x_ref[...]
    ha_ref[...] = lax.dot_general(x, wa_ref[...], (((1,), (1,)), ((), ())),
                                  precision=_PREC,
                                  preferred_element_type=jnp.float32)
    hb_ref[...] = lax.dot_general(x, wb_ref[...], (((1,), (1,)), ((), ())),
                                  precision=_PREC,
                                  preferred_element_type=jnp.float32)


def _tc_mm2(x, wa, wb):
    return pl.pallas_call(
        _mm2_body,
        grid=(_GRID,),
        in_specs=[
            pl.BlockSpec((_BLK, D), lambda i: (i, 0)),
            pl.BlockSpec((D, D), lambda i: (0, 0)),
            pl.BlockSpec((D, D), lambda i: (0, 0)),
        ],
        out_specs=[
            pl.BlockSpec((_BLK, D), lambda i: (i, 0)),
            pl.BlockSpec((_BLK, D), lambda i: (i, 0)),
        ],
        out_shape=[
            jax.ShapeDtypeStruct((N, D), jnp.float32),
            jax.ShapeDtypeStruct((N, D), jnp.float32),
        ],
    )(x, wa, wb)


def _post_mm_body(s_ref, h_ref, dis_ref, b_ref, w_ref, out_ref, *, relu):
    d = dis_ref[...]
    z = d * (s_ref[0] + s_ref[1]) + (d * d) * h_ref[...] + b_ref[...]
    if relu:
        z = jnp.maximum(z, 0.0)
    out_ref[...] = lax.dot_general(z, w_ref[...], (((1,), (1,)), ((), ())),
                                   precision=_PREC,
                                   preferred_element_type=jnp.float32)


def _tc_post_mm(s_parts, h, dis, b, w, relu, d_out):
    return pl.pallas_call(
        functools.partial(_post_mm_body, relu=relu),
        grid=(_GRID,),
        in_specs=[
            pl.BlockSpec((2, _BLK, D), lambda i: (0, i, 0)),
            pl.BlockSpec((_BLK, D), lambda i: (i, 0)),
            pl.BlockSpec((_BLK, 1), lambda i: (i, 0)),
            pl.BlockSpec((1, D), lambda i: (0, 0)),
            pl.BlockSpec((d_out, D), lambda i: (0, 0)),
        ],
        out_specs=pl.BlockSpec((_BLK, d_out), lambda i: (i, 0)),
        out_shape=jax.ShapeDtypeStruct((N, d_out), jnp.float32),
    )(s_parts, h, dis, b, w)


def _xsol_body(s_ref, h_ref, dis_ref, b_ref, wol_ref, bol_ref, hga_ref,
               wg1b_ref, xsol_ref, hg_ref):
    d = dis_ref[...]
    z = d * (s_ref[0] + s_ref[1]) + (d * d) * h_ref[...] + b_ref[...]
    xsol = lax.dot_general(z, wol_ref[...], (((1,), (1,)), ((), ())),
                           precision=_PREC,
                           preferred_element_type=jnp.float32) + bol_ref[...]
    xsol_ref[...] = xsol
    hg_ref[...] = hga_ref[...] + xsol[:, D - 1:D] * wg1b_ref[...]


def _tc_xsol(s_parts, h, dis, b, wol, bol, hga, wg1b):
    return pl.pallas_call(
        _xsol_body,
        grid=(_GRID,),
        in_specs=[
            pl.BlockSpec((2, _BLK, D), lambda i: (0, i, 0)),
            pl.BlockSpec((_BLK, D), lambda i: (i, 0)),
            pl.BlockSpec((_BLK, 1), lambda i: (i, 0)),
            pl.BlockSpec((1, D), lambda i: (0, 0)),
            pl.BlockSpec((D, D), lambda i: (0, 0)),
            pl.BlockSpec((1, D), lambda i: (0, 0)),
            pl.BlockSpec((_BLK, D), lambda i: (i, 0)),
            pl.BlockSpec((1, D), lambda i: (0, 0)),
        ],
        out_specs=[
            pl.BlockSpec((_BLK, D), lambda i: (i, 0)),
            pl.BlockSpec((_BLK, D), lambda i: (i, 0)),
        ],
        out_shape=[
            jax.ShapeDtypeStruct((N, D), jnp.float32),
            jax.ShapeDtypeStruct((N, D), jnp.float32),
        ],
    )(s_parts, h, dis, b, wol, bol, hga, wg1b)


def _final_body(s_ref, h4_ref, dis_ref, consts_ref, xsol_ref, xlast_ref,
                xnew_ref, gamma_ref):
    d = dis_ref[...]
    bg2 = consts_ref[0, 0]
    wgl = consts_ref[0, 1]
    bgl = consts_ref[0, 2]
    g = d * (s_ref[0] + s_ref[1]) + (d * d) * h4_ref[...] + bg2
    gl = g * wgl + bgl
    gmin = jnp.min(gl)
    gmax = jnp.max(gl)
    gamma = (gl - gmin) / (gmax - gmin + 1e-12)
    gamma_ref[...] = gamma
    xsol = xsol_ref[...]
    xl = xlast_ref[...]
    upd = xl + gamma * (xsol[:, D - 1:D] - xl)
    col = lax.broadcasted_iota(jnp.int32, (N, D), 1)
    xnew_ref[...] = jnp.where(col == D - 1, upd, xsol)


def _tc_final(s_parts, h4, dis, consts, xsol, xlast):
    return pl.pallas_call(
        _final_body,
        out_shape=[
            jax.ShapeDtypeStruct((N, D), jnp.float32),
            jax.ShapeDtypeStruct((N, 1), jnp.float32),
        ],
    )(s_parts, h4, dis, consts, xsol, xlast)


def kernel(x, edge_index, edge_weights, Wo1, bo1, Wo2, bo2, Wol, bol,
           Wg1, bg1, Wg2, bg2, Wgl, bgl):
    row = edge_index[0].astype(jnp.int32)
    col = edge_index[1].astype(jnp.int32)
    ew = edge_weights.astype(jnp.float32)

    wg1a = Wg1[:, :D]
    wg1b = Wg1[:, D].reshape(1, D)

    deg_parts = _sc_deg(col, ew)
    dis = _tc_dis(deg_parts)
    h1, hga = _tc_mm2(x, Wo1, wg1a)
    wre = _sc_wre(row, ew, dis.reshape(N))

    s1 = _sc_agg(h1, row, col, wre)
    h2 = _tc_post_mm(s1, h1, dis, bo1.reshape(1, D), Wo2, True, D)

    s2 = _sc_agg(h2, row, col, wre)
    xsol, hg = _tc_xsol(s2, h2, dis, bo2.reshape(1, D), Wol,
                        bol.reshape(1, D), hga, wg1b)

    s3 = _sc_agg(hg, row, col, wre)
    h4 = _tc_post_mm(s3, hg, dis, bg1.reshape(1, D), Wg2, True, 1)

    s4 = _sc_agg1(h4.reshape(N), row, col, wre)

    consts = jnp.stack([bg2[0], Wgl[0, 0], bgl[0]]).reshape(1, 3)
    xnew, gamma = _tc_final(s4.reshape(NC, N, 1), h4, dis, consts,
                            xsol, x[:, D - 1:D])
    return (xnew, gamma)


# speedup vs baseline: 12.6444x; 12.6444x over previous
import functools

import jax
import jax.numpy as jnp
from jax import lax
from jax.experimental import pallas as pl
from jax.experimental.pallas import tpu as pltpu
from jax.experimental.pallas import tpu_sc as plsc

N = 10000
E = 320000
D = 128
NC, NS, L = 2, 16, 16
NW = NC * NS
EPT = E // NW
RPT = 632
NPAD = RPT * NS
K = 80
NCH = EPT // K

_MESH = plsc.VectorSubcoreMesh(core_axis_name="c", subcore_axis_name="s")
_PREC = lax.Precision.HIGHEST


def _worker(base_len):
    cid = lax.axis_index("c")
    sid = lax.axis_index("s")
    wid = sid * NC + cid
    return cid, sid, pl.multiple_of(wid * base_len, 8)


def _zero_shared_rows(z_v, acc_sh, sid):
    nz = z_v.shape[0]
    zero = jnp.zeros((L,), jnp.float32)

    if len(z_v.shape) == 1:
        @pl.loop(0, nz, step=L)
        def _(i):
            z_v[pl.ds(i, L)] = zero
    else:
        @pl.loop(0, nz)
        def _(j):
            for q in range(z_v.shape[1] // L):
                z_v[j, pl.ds(q * L, L)] = zero

    start = pl.multiple_of(sid * RPT, 8)
    off = 0
    while off < RPT:
        step = min(nz, RPT - off)
        pltpu.sync_copy(z_v.at[pl.ds(0, step)], acc_sh.at[pl.ds(start + off, step)])
        off += step


@functools.partial(
    pl.kernel,
    out_type=jax.ShapeDtypeStruct((NC * NPAD,), jnp.float32),
    mesh=_MESH,
    scratch_types=[
        pltpu.VMEM_SHARED((NPAD,), jnp.float32),
        pltpu.VMEM((EPT,), jnp.float32),
        pltpu.VMEM((EPT,), jnp.int32),
        pltpu.VMEM((RPT,), jnp.float32),
    ],
)
def _sc_deg(c_hbm, w_hbm, out_hbm, acc_sh, w_v, c_v, z_v):
    cid, sid, base = _worker(EPT)
    _zero_shared_rows(z_v, acc_sh, sid)
    plsc.subcore_barrier()
    pltpu.sync_copy(w_hbm.at[pl.ds(base, EPT)], w_v)
    pltpu.sync_copy(c_hbm.at[pl.ds(base, EPT)], c_v)
    pltpu.sync_copy(w_v, acc_sh.at[c_v], add=True)
    plsc.subcore_barrier()
    s = pl.multiple_of(sid * RPT, 8)
    d0 = pl.multiple_of(cid * NPAD + sid * RPT, 8)
    pltpu.sync_copy(acc_sh.at[pl.ds(s, RPT)], z_v)
    pltpu.sync_copy(z_v, out_hbm.at[pl.ds(d0, RPT)])


@functools.partial(
    pl.kernel,
    out_type=jax.ShapeDtypeStruct((E,), jnp.float32),
    mesh=_MESH,
    scratch_types=[
        pltpu.VMEM((EPT,), jnp.int32),
        pltpu.VMEM((EPT,), jnp.float32),
        pltpu.VMEM((EPT,), jnp.float32),
        pltpu.SemaphoreType.DMA,
    ],
)
def _sc_wre(r_hbm, ew_hbm, dis_hbm, out_hbm, r_v, d_v, w_v, sem):
    _, _, base = _worker(EPT)
    pltpu.sync_copy(r_hbm.at[pl.ds(base, EPT)], r_v)
    pltpu.sync_copy(ew_hbm.at[pl.ds(base, EPT)], w_v)
    pltpu.async_copy(dis_hbm.at[r_v], d_v, sem).wait()

    @pl.loop(0, EPT, step=L)
    def _(i):
        w_v[pl.ds(i, L)] = w_v[pl.ds(i, L)] * d_v[pl.ds(i, L)]

    pltpu.sync_copy(w_v, out_hbm.at[pl.ds(base, EPT)])


@functools.partial(
    pl.kernel,
    out_type=jax.ShapeDtypeStruct((NC * NPAD, D), jnp.float32),
    mesh=_MESH,
    scratch_types=[
        pltpu.VMEM_SHARED((NPAD, D), jnp.float32),
        pltpu.VMEM((K, D), jnp.float32),
        pltpu.VMEM((K, D), jnp.float32),
        pltpu.VMEM((K,), jnp.int32),
        pltpu.VMEM((K,), jnp.int32),
        pltpu.VMEM((K,), jnp.int32),
        pltpu.VMEM((K,), jnp.int32),
        pltpu.VMEM((K,), jnp.float32),
        pltpu.VMEM((K,), jnp.float32),
        pltpu.SemaphoreType.DMA,
        pltpu.SemaphoreType.DMA,
    ],
)
def _sc_agg(h_hbm, r_hbm, c_hbm, w_hbm, out_hbm, acc_sh,
            rows0, rows1, r0, r1, c0, c1, w0, w1, sem0, sem1):
    cid, sid, base = _worker(EPT)
    _zero_shared_rows(rows0, acc_sh, sid)
    plsc.subcore_barrier()

    bufs = ((rows0, r0, c0, w0, sem0), (rows1, r1, c1, w1, sem1))

    def start(k, buf):
        rows, r_v, c_v, w_v, sem = buf
        off = pl.multiple_of(base + k * K, 8)
        pltpu.sync_copy(r_hbm.at[pl.ds(off, K)], r_v)
        pltpu.sync_copy(c_hbm.at[pl.ds(off, K)], c_v)
        pltpu.sync_copy(w_hbm.at[pl.ds(off, K)], w_v)
        return pltpu.async_copy(h_hbm.at[r_v], rows, sem)

    def process(buf):
        rows, r_v, c_v, w_v, sem = buf
        pltpu.make_async_copy(h_hbm.at[r_v], rows, sem).wait()

        @pl.loop(0, K, step=L)
        def _(j0):
            wv = w_v[pl.ds(j0, L)]
            for t in range(L):
                s = wv[t]
                for q in range(D // L):
                    rows[j0 + t, pl.ds(q * L, L)] = rows[j0 + t, pl.ds(q * L, L)] * s

        pltpu.sync_copy(rows, acc_sh.at[c_v], add=True)

    start(0, bufs[0])
    start(1, bufs[1])

    @pl.loop(0, NCH - 1, step=2)
    def _(k):
        for b in range(2):
            process(bufs[b])

            @pl.when(k + b + 2 < NCH)
            def _():
                start(k + b + 2, bufs[b])

    process(bufs[(NCH - 1) % 2])

    plsc.subcore_barrier()
    s0 = pl.multiple_of(sid * RPT, 8)
    d0 = pl.multiple_of(cid * NPAD + sid * RPT, 8)
    off = 0
    while off < RPT:
        step = min(K, RPT - off)
        pltpu.sync_copy(acc_sh.at[pl.ds(s0 + off, step)], rows0.at[pl.ds(0, step)])
        pltpu.sync_copy(rows0.at[pl.ds(0, step)], out_hbm.at[pl.ds(d0 + off, step)])
        off += step


@functools.partial(
    pl.kernel,
    out_type=jax.ShapeDtypeStruct((NC * NPAD,), jnp.float32),
    mesh=_MESH,
    scratch_types=[
        pltpu.VMEM_SHARED((NPAD,), jnp.float32),
        pltpu.VMEM((EPT,), jnp.int32),
        pltpu.VMEM((EPT,), jnp.int32),
        pltpu.VMEM((EPT,), jnp.float32),
        pltpu.VMEM((EPT,), jnp.float32),
        pltpu.VMEM((RPT,), jnp.float32),
        pltpu.SemaphoreType.DMA,
    ],
)
def _sc_agg1(g_hbm, r_hbm, c_hbm, w_hbm, out_hbm, acc_sh,
             r_v, c_v, w_v, g_v, z_v, sem):
    cid, sid, base = _worker(EPT)
    _zero_shared_rows(z_v, acc_sh, sid)
    plsc.subcore_barrier()
    pltpu.sync_copy(r_hbm.at[pl.ds(base, EPT)], r_v)
    pltpu.sync_copy(c_hbm.at[pl.ds(base, EPT)], c_v)
    pltpu.sync_copy(w_hbm.at[pl.ds(base, EPT)], w_v)
    pltpu.async_copy(g_hbm.at[r_v], g_v, sem).wait()

    @pl.loop(0, EPT, step=L)
    def _(i):
        g_v[pl.ds(i, L)] = g_v[pl.ds(i, L)] * w_v[pl.ds(i, L)]

    pltpu.sync_copy(g_v, acc_sh.at[c_v], add=True)
    plsc.subcore_barrier()
    s = pl.multiple_of(sid * RPT, 8)
    d0 = pl.multiple_of(cid * NPAD + sid * RPT, 8)
    pltpu.sync_copy(acc_sh.at[pl.ds(s, RPT)], z_v)
    pltpu.sync_copy(z_v, out_hbm.at[pl.ds(d0, RPT)])


_BLK = 1000
_GRID = N // _BLK


def _dis_body(deg_ref, dis_ref):
    d = deg_ref[0, :] + deg_ref[1, :] + 1.0
    dis_ref[...] = lax.rsqrt(d)


def _tc_dis(deg_parts):
    return pl.pallas_call(
        _dis_body,
        out_shape=jax.ShapeDtypeStruct((NPAD,), jnp.float32),
    )(deg_parts)


def _mm2_body(x_ref, wa_ref, wb_ref, ha_ref, hb_ref):
    x = x_ref[...]
    ha_ref[...] = lax.dot_general(x, wa_ref[...], (((1,), (1,)), ((), ())),
                                  precision=_PREC,
                                  preferred_element_type=jnp.float32)
    hb_ref[...] = lax.dot_general(x, wb_ref[...], (((1,), (1,)), ((), ())),
                                  precision=_PREC,
                                  preferred_element_type=jnp.float32)


def _tc_mm2(x, wa, wb):
    return pl.pallas_call(
        _mm2_body,
        grid=(_GRID,),
        in_specs=[
            pl.BlockSpec((_BLK, D), lambda i: (i, 0)),
            pl.BlockSpec((D, D), lambda i: (0, 0)),
            pl.BlockSpec((D, D), lambda i: (0, 0)),
        ],
        out_specs=[
            pl.BlockSpec((_BLK, D), lambda i: (i, 0)),
            pl.BlockSpec((_BLK, D), lambda i: (i, 0)),
        ],
        out_shape=[
            jax.ShapeDtypeStruct((N, D), jnp.float32),
            jax.ShapeDtypeStruct((N, D), jnp.float32),
        ],
    )(x, wa, wb)


def _post_mm_body(s_ref, h_ref, dis_ref, b_ref, w_ref, out_ref, *, relu):
    d = dis_ref[...]
    z = d * (s_ref[0] + s_ref[1]) + (d * d) * h_ref[...] + b_ref[...]
    if relu:
        z = jnp.maximum(z, 0.0)
    out_ref[...] = lax.dot_general(z, w_ref[...], (((1,), (1,)), ((), ())),
                                   precision=_PREC,
                                   preferred_element_type=jnp.float32)


def _tc_post_mm(s_parts, h, dis, b, w, relu, d_out):
    return pl.pallas_call(
        functools.partial(_post_mm_body, relu=relu),
        grid=(_GRID,),
        in_specs=[
            pl.BlockSpec((2, _BLK, D), lambda i: (0, i, 0)),
            pl.BlockSpec((_BLK, D), lambda i: (i, 0)),
            pl.BlockSpec((_BLK, 1), lambda i: (i, 0)),
            pl.BlockSpec((1, D), lambda i: (0, 0)),
            pl.BlockSpec((d_out, D), lambda i: (0, 0)),
        ],
        out_specs=pl.BlockSpec((_BLK, d_out), lambda i: (i, 0)),
        out_shape=jax.ShapeDtypeStruct((N, d_out), jnp.float32),
    )(s_parts, h, dis, b, w)


def _xsol_body(s_ref, h_ref, dis_ref, b_ref, wol_ref, bol_ref, hga_ref,
               wg1b_ref, xsol_ref, hg_ref):
    d = dis_ref[...]
    z = d * (s_ref[0] + s_ref[1]) + (d * d) * h_ref[...] + b_ref[...]
    xsol = lax.dot_general(z, wol_ref[...], (((1,), (1,)), ((), ())),
                           precision=_PREC,
                           preferred_element_type=jnp.float32) + bol_ref[...]
    xsol_ref[...] = xsol
    hg_ref[...] = hga_ref[...] + xsol[:, D - 1:D] * wg1b_ref[...]


def _tc_xsol(s_parts, h, dis, b, wol, bol, hga, wg1b):
    return pl.pallas_call(
        _xsol_body,
        grid=(_GRID,),
        in_specs=[
            pl.BlockSpec((2, _BLK, D), lambda i: (0, i, 0)),
            pl.BlockSpec((_BLK, D), lambda i: (i, 0)),
            pl.BlockSpec((_BLK, 1), lambda i: (i, 0)),
            pl.BlockSpec((1, D), lambda i: (0, 0)),
            pl.BlockSpec((D, D), lambda i: (0, 0)),
            pl.BlockSpec((1, D), lambda i: (0, 0)),
            pl.BlockSpec((_BLK, D), lambda i: (i, 0)),
            pl.BlockSpec((1, D), lambda i: (0, 0)),
        ],
        out_specs=[
            pl.BlockSpec((_BLK, D), lambda i: (i, 0)),
            pl.BlockSpec((_BLK, D), lambda i: (i, 0)),
        ],
        out_shape=[
            jax.ShapeDtypeStruct((N, D), jnp.float32),
            jax.ShapeDtypeStruct((N, D), jnp.float32),
        ],
    )(s_parts, h, dis, b, wol, bol, hga, wg1b)


def _final_body(s_ref, h4_ref, dis_ref, consts_ref, xsol_ref, xlast_ref,
                xnew_ref, gamma_ref):
    d = dis_ref[...]
    bg2 = consts_ref[0, 0]
    wgl = consts_ref[0, 1]
    bgl = consts_ref[0, 2]
    g = d * (s_ref[0] + s_ref[1]) + (d * d) * h4_ref[...] + bg2
    gl = g * wgl + bgl
    gmin = jnp.min(gl)
    gmax = jnp.max(gl)
    gamma = (gl - gmin) / (gmax - gmin + 1e-12)
    gamma_ref[...] = gamma
    xsol = xsol_ref[...]
    xl = xlast_ref[...]
    upd = xl + gamma * (xsol[:, D - 1:D] - xl)
    col = lax.broadcasted_iota(jnp.int32, (N, D), 1)
    xnew_ref[...] = jnp.where(col == D - 1, upd, xsol)


def _tc_final(s_parts, h4, dis, consts, xsol, xlast):
    return pl.pallas_call(
        _final_body,
        out_shape=[
            jax.ShapeDtypeStruct((N, D), jnp.float32),
            jax.ShapeDtypeStruct((N, 1), jnp.float32),
        ],
    )(s_parts, h4, dis, consts, xsol, xlast)


def kernel(x, edge_index, edge_weights, Wo1, bo1, Wo2, bo2, Wol, bol,
           Wg1, bg1, Wg2, bg2, Wgl, bgl):
    row = edge_index[0].astype(jnp.int32)
    col = edge_index[1].astype(jnp.int32)
    ew = edge_weights.astype(jnp.float32)

    wg1a = Wg1[:, :D]
    wg1b = Wg1[:, D].reshape(1, D)

    deg_parts = _sc_deg(col, ew).reshape(NC, NPAD)
    dis1 = _tc_dis(deg_parts)[:N]
    dis = dis1.reshape(N, 1)
    h1, hga = _tc_mm2(x, Wo1, wg1a)
    wre = _sc_wre(row, ew, dis1)

    s1 = _sc_agg(h1, row, col, wre).reshape(NC, NPAD, D)
    h2 = _tc_post_mm(s1, h1, dis, bo1.reshape(1, D), Wo2, True, D)

    s2 = _sc_agg(h2, row, col, wre).reshape(NC, NPAD, D)
    xsol, hg = _tc_xsol(s2, h2, dis, bo2.reshape(1, D), Wol,
                        bol.reshape(1, D), hga, wg1b)

    s3 = _sc_agg(hg, row, col, wre).reshape(NC, NPAD, D)
    h4 = _tc_post_mm(s3, hg, dis, bg1.reshape(1, D), Wg2, True, 1)

    s4 = _sc_agg1(h4.reshape(N), row, col, wre).reshape(NC, NPAD)[:, :N]

    consts = jnp.stack([bg2[0], Wgl[0, 0], bgl[0]]).reshape(1, 3)
    xnew, gamma = _tc_final(s4.reshape(NC, N, 1), h4, dis, consts,
                            xsol, x[:, D - 1:D])
    return (xnew, gamma)


# speedup vs baseline: 20.5027x; 1.6215x over previous
import functools

import jax
import jax.numpy as jnp
from jax import lax
from jax.experimental import pallas as pl
from jax.experimental.pallas import tpu as pltpu
from jax.experimental.pallas import tpu_sc as plsc

N = 10000
E = 320000
D = 128
NC, NS, L = 2, 16, 16
NW = NC * NS
EPT = E // NW
RPT = 632
NPAD = RPT * NS
K = 80
NCH = EPT // K

_MESH = plsc.VectorSubcoreMesh(core_axis_name="c", subcore_axis_name="s")
_PREC = lax.Precision.HIGHEST


def _worker(base_len):
    cid = lax.axis_index("c")
    sid = lax.axis_index("s")
    wid = sid * NC + cid
    return cid, sid, pl.multiple_of(wid * base_len, 8)


def _zero_shared_rows(z_v, acc_sh, sid):
    nz = z_v.shape[0]
    zero = jnp.zeros((L,), jnp.float32)

    if len(z_v.shape) == 1:
        @pl.loop(0, nz, step=L)
        def _(i):
            z_v[pl.ds(i, L)] = zero
    else:
        @pl.loop(0, nz)
        def _(j):
            for q in range(z_v.shape[1] // L):
                z_v[j, pl.ds(q * L, L)] = zero

    start = pl.multiple_of(sid * RPT, 8)
    off = 0
    while off < RPT:
        step = min(nz, RPT - off)
        pltpu.sync_copy(z_v.at[pl.ds(0, step)], acc_sh.at[pl.ds(start + off, step)])
        off += step


@functools.partial(
    pl.kernel,
    out_type=jax.ShapeDtypeStruct((NC * NPAD,), jnp.float32),
    mesh=_MESH,
    scratch_types=[
        pltpu.VMEM_SHARED((NPAD,), jnp.float32),
        pltpu.VMEM((EPT,), jnp.float32),
        pltpu.VMEM((EPT,), jnp.int32),
        pltpu.VMEM((RPT,), jnp.float32),
    ],
)
def _sc_deg(c_hbm, w_hbm, out_hbm, acc_sh, w_v, c_v, z_v):
    cid, sid, base = _worker(EPT)
    _zero_shared_rows(z_v, acc_sh, sid)
    plsc.subcore_barrier()
    pltpu.sync_copy(w_hbm.at[pl.ds(base, EPT)], w_v)
    pltpu.sync_copy(c_hbm.at[pl.ds(base, EPT)], c_v)
    pltpu.sync_copy(w_v, acc_sh.at[c_v], add=True)
    plsc.subcore_barrier()
    s = pl.multiple_of(sid * RPT, 8)
    d0 = pl.multiple_of(cid * NPAD + sid * RPT, 8)
    pltpu.sync_copy(acc_sh.at[pl.ds(s, RPT)], z_v)
    pltpu.sync_copy(z_v, out_hbm.at[pl.ds(d0, RPT)])


@functools.partial(
    pl.kernel,
    out_type=jax.ShapeDtypeStruct((E,), jnp.float32),
    mesh=_MESH,
    scratch_types=[
        pltpu.VMEM((EPT,), jnp.int32),
        pltpu.VMEM((EPT,), jnp.float32),
        pltpu.VMEM((EPT,), jnp.float32),
        pltpu.SemaphoreType.DMA,
    ],
)
def _sc_wre(r_hbm, ew_hbm, dis_hbm, out_hbm, r_v, d_v, w_v, sem):
    _, _, base = _worker(EPT)
    pltpu.sync_copy(r_hbm.at[pl.ds(base, EPT)], r_v)
    pltpu.sync_copy(ew_hbm.at[pl.ds(base, EPT)], w_v)
    pltpu.async_copy(dis_hbm.at[r_v], d_v, sem).wait()

    @pl.loop(0, EPT, step=L)
    def _(i):
        w_v[pl.ds(i, L)] = w_v[pl.ds(i, L)] * d_v[pl.ds(i, L)]

    pltpu.sync_copy(w_v, out_hbm.at[pl.ds(base, EPT)])


@functools.partial(
    pl.kernel,
    out_type=jax.ShapeDtypeStruct((NC * NPAD, D), jnp.float32),
    mesh=_MESH,
    scratch_types=[
        pltpu.VMEM_SHARED((NPAD, D), jnp.float32),
        pltpu.VMEM((K, D), jnp.float32),
        pltpu.VMEM((K, D), jnp.float32),
        pltpu.VMEM((EPT,), jnp.int32),
        pltpu.VMEM((EPT,), jnp.float32),
        pltpu.VMEM((K,), jnp.int32),
        pltpu.VMEM((K,), jnp.int32),
        pltpu.VMEM((K,), jnp.int32),
        pltpu.VMEM((K,), jnp.int32),
        pltpu.SemaphoreType.DMA,
        pltpu.SemaphoreType.DMA,
    ],
)
def _sc_agg(h_hbm, rc_hbm, w_hbm, out_hbm, acc_sh,
            rows0, rows1, rc_all, w_all, r0, r1, c0, c1, sem0, sem1):
    cid, sid, base = _worker(EPT)
    _zero_shared_rows(rows0, acc_sh, sid)
    pltpu.sync_copy(rc_hbm.at[pl.ds(base, EPT)], rc_all)
    pltpu.sync_copy(w_hbm.at[pl.ds(base, EPT)], w_all)
    plsc.subcore_barrier()

    bufs = ((rows0, r0, c0, sem0), (rows1, r1, c1, sem1))

    def start(k, buf):
        rows, r_v, c_v, sem = buf
        off = pl.multiple_of(k * K, 8)
        for j in range(0, K, L):
            rc = rc_all[pl.ds(off + j, L)]
            r_v[pl.ds(j, L)] = lax.shift_right_logical(rc, 16)
            c_v[pl.ds(j, L)] = lax.bitwise_and(rc, jnp.int32(0xFFFF))
        return pltpu.async_copy(h_hbm.at[r_v], rows, sem)

    def process(k, buf):
        rows, r_v, c_v, sem = buf
        pltpu.make_async_copy(h_hbm.at[r_v], rows, sem).wait()
        woff = pl.multiple_of(k * K, 8)

        @pl.loop(0, K, step=L)
        def _(j0):
            wv = w_all[pl.ds(woff + j0, L)]
            for t in range(L):
                s = wv[t]
                for q in range(D // L):
                    rows[j0 + t, pl.ds(q * L, L)] = rows[j0 + t, pl.ds(q * L, L)] * s

        pltpu.sync_copy(rows, acc_sh.at[c_v], add=True)

    start(0, bufs[0])
    start(1, bufs[1])

    @pl.loop(0, NCH - 1, step=2)
    def _(k):
        for b in range(2):
            process(k + b, bufs[b])

            @pl.when(k + b + 2 < NCH)
            def _():
                start(k + b + 2, bufs[b])

    process(NCH - 1, bufs[(NCH - 1) % 2])

    plsc.subcore_barrier()
    s0 = pl.multiple_of(sid * RPT, 8)
    d0 = pl.multiple_of(cid * NPAD + sid * RPT, 8)
    off = 0
    while off < RPT:
        step = min(K, RPT - off)
        pltpu.sync_copy(acc_sh.at[pl.ds(s0 + off, step)], rows0.at[pl.ds(0, step)])
        pltpu.sync_copy(rows0.at[pl.ds(0, step)], out_hbm.at[pl.ds(d0 + off, step)])
        off += step


@functools.partial(
    pl.kernel,
    out_type=jax.ShapeDtypeStruct((NC * NPAD,), jnp.float32),
    mesh=_MESH,
    scratch_types=[
        pltpu.VMEM_SHARED((NPAD,), jnp.float32),
        pltpu.VMEM((EPT,), jnp.int32),
        pltpu.VMEM((EPT,), jnp.int32),
        pltpu.VMEM((EPT,), jnp.float32),
        pltpu.VMEM((EPT,), jnp.float32),
        pltpu.VMEM((RPT,), jnp.float32),
        pltpu.SemaphoreType.DMA,
    ],
)
def _sc_agg1(g_hbm, r_hbm, c_hbm, w_hbm, out_hbm, acc_sh,
             r_v, c_v, w_v, g_v, z_v, sem):
    cid, sid, base = _worker(EPT)
    _zero_shared_rows(z_v, acc_sh, sid)
    plsc.subcore_barrier()
    pltpu.sync_copy(r_hbm.at[pl.ds(base, EPT)], r_v)
    pltpu.sync_copy(c_hbm.at[pl.ds(base, EPT)], c_v)
    pltpu.sync_copy(w_hbm.at[pl.ds(base, EPT)], w_v)
    pltpu.async_copy(g_hbm.at[r_v], g_v, sem).wait()

    @pl.loop(0, EPT, step=L)
    def _(i):
        g_v[pl.ds(i, L)] = g_v[pl.ds(i, L)] * w_v[pl.ds(i, L)]

    pltpu.sync_copy(g_v, acc_sh.at[c_v], add=True)
    plsc.subcore_barrier()
    s = pl.multiple_of(sid * RPT, 8)
    d0 = pl.multiple_of(cid * NPAD + sid * RPT, 8)
    pltpu.sync_copy(acc_sh.at[pl.ds(s, RPT)], z_v)
    pltpu.sync_copy(z_v, out_hbm.at[pl.ds(d0, RPT)])


_BLK = 1000
_GRID = N // _BLK


def _dis_body(deg_ref, dis_ref):
    d = deg_ref[0, :] + deg_ref[1, :] + 1.0
    dis_ref[...] = lax.rsqrt(d)


def _tc_dis(deg_parts):
    return pl.pallas_call(
        _dis_body,
        out_shape=jax.ShapeDtypeStruct((NPAD,), jnp.float32),
    )(deg_parts)


def _mm2_body(x_ref, wa_ref, wb_ref, ha_ref, hb_ref):
    x = x_ref[...]
    ha_ref[...] = lax.dot_general(x, wa_ref[...], (((1,), (1,)), ((), ())),
                                  precision=_PREC,
                                  preferred_element_type=jnp.float32)
    hb_ref[...] = lax.dot_general(x, wb_ref[...], (((1,), (1,)), ((), ())),
                                  precision=_PREC,
                                  preferred_element_type=jnp.float32)


def _tc_mm2(x, wa, wb):
    return pl.pallas_call(
        _mm2_body,
        grid=(_GRID,),
        in_specs=[
            pl.BlockSpec((_BLK, D), lambda i: (i, 0)),
            pl.BlockSpec((D, D), lambda i: (0, 0)),
            pl.BlockSpec((D, D), lambda i: (0, 0)),
        ],
        out_specs=[
            pl.BlockSpec((_BLK, D), lambda i: (i, 0)),
            pl.BlockSpec((_BLK, D), lambda i: (i, 0)),
        ],
        out_shape=[
            jax.ShapeDtypeStruct((N, D), jnp.float32),
            jax.ShapeDtypeStruct((N, D), jnp.float32),
        ],
    )(x, wa, wb)


def _post_mm_body(s_ref, h_ref, dis_ref, b_ref, w_ref, out_ref, *, relu):
    d = dis_ref[...]
    z = d * (s_ref[0] + s_ref[1]) + (d * d) * h_ref[...] + b_ref[...]
    if relu:
        z = jnp.maximum(z, 0.0)
    out_ref[...] = lax.dot_general(z, w_ref[...], (((1,), (1,)), ((), ())),
                                   precision=_PREC,
                                   preferred_element_type=jnp.float32)


def _tc_post_mm(s_parts, h, dis, b, w, relu, d_out):
    return pl.pallas_call(
        functools.partial(_post_mm_body, relu=relu),
        grid=(_GRID,),
        in_specs=[
            pl.BlockSpec((2, _BLK, D), lambda i: (0, i, 0)),
            pl.BlockSpec((_BLK, D), lambda i: (i, 0)),
            pl.BlockSpec((_BLK, 1), lambda i: (i, 0)),
            pl.BlockSpec((1, D), lambda i: (0, 0)),
            pl.BlockSpec((d_out, D), lambda i: (0, 0)),
        ],
        out_specs=pl.BlockSpec((_BLK, d_out), lambda i: (i, 0)),
        out_shape=jax.ShapeDtypeStruct((N, d_out), jnp.float32),
    )(s_parts, h, dis, b, w)


def _xsol_body(s_ref, h_ref, dis_ref, b_ref, wol_ref, bol_ref, hga_ref,
               wg1b_ref, xsol_ref, hg_ref):
    d = dis_ref[...]
    z = d * (s_ref[0] + s_ref[1]) + (d * d) * h_ref[...] + b_ref[...]
    xsol = lax.dot_general(z, wol_ref[...], (((1,), (1,)), ((), ())),
                           precision=_PREC,
                           preferred_element_type=jnp.float32) + bol_ref[...]
    xsol_ref[...] = xsol
    hg_ref[...] = hga_ref[...] + xsol[:, D - 1:D] * wg1b_ref[...]


def _tc_xsol(s_parts, h, dis, b, wol, bol, hga, wg1b):
    return pl.pallas_call(
        _xsol_body,
        grid=(_GRID,),
        in_specs=[
            pl.BlockSpec((2, _BLK, D), lambda i: (0, i, 0)),
            pl.BlockSpec((_BLK, D), lambda i: (i, 0)),
            pl.BlockSpec((_BLK, 1), lambda i: (i, 0)),
            pl.BlockSpec((1, D), lambda i: (0, 0)),
            pl.BlockSpec((D, D), lambda i: (0, 0)),
            pl.BlockSpec((1, D), lambda i: (0, 0)),
            pl.BlockSpec((_BLK, D), lambda i: (i, 0)),
            pl.BlockSpec((1, D), lambda i: (0, 0)),
        ],
        out_specs=[
            pl.BlockSpec((_BLK, D), lambda i: (i, 0)),
            pl.BlockSpec((_BLK, D), lambda i: (i, 0)),
        ],
        out_shape=[
            jax.ShapeDtypeStruct((N, D), jnp.float32),
            jax.ShapeDtypeStruct((N, D), jnp.float32),
        ],
    )(s_parts, h, dis, b, wol, bol, hga, wg1b)


def _final_body(s_ref, h4_ref, dis_ref, consts_ref, xsol_ref, xlast_ref,
                xnew_ref, gamma_ref):
    d = dis_ref[...]
    bg2 = consts_ref[0, 0]
    wgl = consts_ref[0, 1]
    bgl = consts_ref[0, 2]
    g = d * (s_ref[0] + s_ref[1]) + (d * d) * h4_ref[...] + bg2
    gl = g * wgl + bgl
    gmin = jnp.min(gl)
    gmax = jnp.max(gl)
    gamma = (gl - gmin) / (gmax - gmin + 1e-12)
    gamma_ref[...] = gamma
    xsol = xsol_ref[...]
    xl = xlast_ref[...]
    upd = xl + gamma * (xsol[:, D - 1:D] - xl)
    col = lax.broadcasted_iota(jnp.int32, (N, D), 1)
    xnew_ref[...] = jnp.where(col == D - 1, upd, xsol)


def _tc_final(s_parts, h4, dis, consts, xsol, xlast):
    return pl.pallas_call(
        _final_body,
        out_shape=[
            jax.ShapeDtypeStruct((N, D), jnp.float32),
            jax.ShapeDtypeStruct((N, 1), jnp.float32),
        ],
    )(s_parts, h4, dis, consts, xsol, xlast)


def kernel(x, edge_index, edge_weights, Wo1, bo1, Wo2, bo2, Wol, bol,
           Wg1, bg1, Wg2, bg2, Wgl, bgl):
    row = edge_index[0].astype(jnp.int32)
    col = edge_index[1].astype(jnp.int32)
    ew = edge_weights.astype(jnp.float32)

    wg1a = Wg1[:, :D]
    wg1b = Wg1[:, D].reshape(1, D)
    rc = jnp.bitwise_or(jnp.left_shift(row, 16), col)

    deg_parts = _sc_deg(col, ew).reshape(NC, NPAD)
    dis1 = _tc_dis(deg_parts)[:N]
    dis = dis1.reshape(N, 1)
    h1, hga = _tc_mm2(x, Wo1, wg1a)
    wre = _sc_wre(row, ew, dis1)

    s1 = _sc_agg(h1, rc, wre).reshape(NC, NPAD, D)
    h2 = _tc_post_mm(s1, h1, dis, bo1.reshape(1, D), Wo2, True, D)

    s2 = _sc_agg(h2, rc, wre).reshape(NC, NPAD, D)
    xsol, hg = _tc_xsol(s2, h2, dis, bo2.reshape(1, D), Wol,
                        bol.reshape(1, D), hga, wg1b)

    s3 = _sc_agg(hg, rc, wre).reshape(NC, NPAD, D)
    h4 = _tc_post_mm(s3, hg, dis, bg1.reshape(1, D), Wg2, True, 1)

    s4 = _sc_agg1(h4.reshape(N), row, col, wre).reshape(NC, NPAD)[:, :N]

    consts = jnp.stack([bg2[0], Wgl[0, 0], bgl[0]]).reshape(1, 3)
    xnew, gamma = _tc_final(s4.reshape(NC, N, 1), h4, dis, consts,
                            xsol, x[:, D - 1:D])
    return (xnew, gamma)
